# baseline (speedup 1.0000x reference)
import functools

import jax
import jax.numpy as jnp
from jax import lax
from jax.experimental import pallas as pl
from jax.experimental.pallas import tpu as pltpu
from jax.experimental.pallas import tpu_sc as plsc

_NSLICES = 2
_LANES = 16


def _make_sc_gather(d, n_slice, flat_base):
    info = plsc.get_sparse_core_info()
    nw = info.num_cores * info.num_subcores
    nc = info.num_cores
    t_per_w = n_slice // nw
    tc = min(32, t_per_w)
    n_chunks = t_per_w // tc
    dh = d // 2
    n_groups = dh // _LANES

    mesh = plsc.VectorSubcoreMesh(core_axis_name="c", subcore_axis_name="s")

    @functools.partial(
        pl.kernel,
        mesh=mesh,
        compiler_params=pltpu.CompilerParams(needs_layout_passes=False),
        out_type=jax.ShapeDtypeStruct((n_slice, dh), jnp.int32),
        scratch_types=[
            pltpu.VMEM((tc,), jnp.int32),
            pltpu.VMEM((tc,), jnp.int32),
            pltpu.VMEM((tc, d), jnp.float32),
            pltpu.VMEM((tc, d), jnp.float32),
            pltpu.VMEM((tc, dh), jnp.int32),
            pltpu.VMEM((tc, dh), jnp.int32),
            pltpu.SemaphoreType.DMA,
            pltpu.SemaphoreType.DMA,
            pltpu.SemaphoreType.DMA,
            pltpu.SemaphoreType.DMA,
        ],
    )
    def gather_kernel(ids_hbm, word_hbm, out_hbm,
                      idx0, idx1, f0, f1, p0, p1,
                      gsem0, gsem1, wsem0, wsem1):
        wid = lax.axis_index("s") * nc + lax.axis_index("c")
        base = wid * t_per_w
        idx = (idx0, idx1)
        fbuf = (f0, f1)
        pbuf = (p0, p1)
        gsem = (gsem0, gsem1)
        wsem = (wsem0, wsem1)

        def pack_chunk(fb, pb):
            def row(r, _):
                for g in range(n_groups):
                    a = plsc.bitcast(fb[r, pl.ds(g * _LANES, _LANES)],
                                     jnp.int32)
                    bb = plsc.bitcast(fb[r, pl.ds(dh + g * _LANES, _LANES)],
                                      jnp.int32)
                    ar = a + 0x7FFF + ((a >> 16) & 1)
                    br = bb + 0x7FFF + ((bb >> 16) & 1)
                    pb[r, pl.ds(g * _LANES, _LANES)] = (
                        ((ar >> 16) & 0xFFFF) | (br & jnp.int32(-65536)))
                return _
            lax.fori_loop(0, tc, row, 0)

        pltpu.sync_copy(ids_hbm.at[pl.ds(flat_base + base, tc)], idx[0])
        gathers = [pltpu.async_copy(word_hbm.at[idx[0]], fbuf[0], gsem[0])]
        writes = []
        for c in range(n_chunks):
            s = c % 2
            if c + 1 < n_chunks:
                sn = (c + 1) % 2
                pltpu.sync_copy(
                    ids_hbm.at[pl.ds(flat_base + base + (c + 1) * tc, tc)],
                    idx[sn])
                gathers.append(
                    pltpu.async_copy(word_hbm.at[idx[sn]], fbuf[sn], gsem[sn]))
            gathers[c].wait()
            if c >= 2:
                writes[c - 2].wait()
            pack_chunk(fbuf[s], pbuf[s])
            writes.append(
                pltpu.async_copy(
                    pbuf[s], out_hbm.at[pl.ds(base + c * tc, tc)], wsem[s]))
        for w in writes[max(0, n_chunks - 2):]:
            w.wait()

    return gather_kernel


def _ln_body(g_ref, p_ref, tt_ref, te_ref, gamma_ref, beta_ref, *rest):
    o_ref = rest[-1]
    gw = g_ref[...]
    lo = lax.bitcast_convert_type(gw << 16, jnp.float32)
    hi = lax.bitcast_convert_type(gw & jnp.int32(-65536), jnp.float32)
    g = jnp.concatenate([lo, hi], axis=-1)
    p = p_ref[...]
    t = tt_ref[...]
    te = te_ref[...]
    h = g + p[None] + te[0:1, :] + t * (te[1:2, :] - te[0:1, :])
    mu = jnp.mean(h, axis=-1, keepdims=True)
    var = jnp.mean((h - mu) ** 2, axis=-1, keepdims=True)
    o_ref[...] = ((h - mu) * lax.rsqrt(var + 1e-5)) * gamma_ref[...] + beta_ref[...]


def _sum_layernorm(gathered, pos_emb, tt_f, type_emb, gamma, beta, prev,
                   si, b, bs, blk):
    _, s, dh = gathered.shape
    d = dh * 2
    grid = (s // blk,)
    in_specs = [
        pl.BlockSpec((bs, blk, dh), lambda i: (0, i, 0)),
        pl.BlockSpec((blk, d), lambda i: (i, 0)),
        pl.BlockSpec((bs, blk, 1), lambda i, _si=si: (_si, i, 0)),
        pl.BlockSpec((2, d), lambda i: (0, 0)),
        pl.BlockSpec((1, d), lambda i: (0, 0)),
        pl.BlockSpec((1, d), lambda i: (0, 0)),
    ]
    args = [gathered, pos_emb, tt_f, type_emb, gamma, beta]
    aliases = {}
    if prev is not None:
        in_specs.append(pl.BlockSpec(memory_space=pl.ANY))
        args.append(prev)
        aliases = {6: 0}
    return pl.pallas_call(
        _ln_body,
        grid=grid,
        in_specs=in_specs,
        out_specs=pl.BlockSpec((bs, blk, d), lambda i, _si=si: (_si, i, 0)),
        out_shape=jax.ShapeDtypeStruct((b, s, d), jnp.float32),
        input_output_aliases=aliases,
    )(*args)


def kernel(input_ids, token_type_ids, word_emb, pos_emb, type_emb, gamma, beta):
    b, s = input_ids.shape
    vocab, d = word_emb.shape
    n = b * s
    bs = b // _NSLICES
    n_slice = bs * s

    ids_flat = input_ids.reshape(n)
    tt_f = token_type_ids.reshape(b, s, 1).astype(jnp.float32)
    gamma2 = gamma.reshape(1, d)
    beta2 = beta.reshape(1, d)

    gathered = [
        _make_sc_gather(d, n_slice, si * n_slice)(ids_flat, word_emb)
        for si in range(_NSLICES)
    ]

    out = None
    for si in range(_NSLICES):
        out = _sum_layernorm(
            gathered[si].reshape(bs, s, d // 2), pos_emb, tt_f, type_emb,
            gamma2, beta2, out, si, b, bs, blk=512,
        )
    return out


# speedup vs baseline: 1.4407x; 1.4407x over previous
import functools

import jax
import jax.numpy as jnp
from jax import lax
from jax.experimental import pallas as pl
from jax.experimental.pallas import tpu as pltpu
from jax.experimental.pallas import tpu_sc as plsc

_NSLICES = 2
_LANES = 16


def _make_sc_gather(d, n_slice, flat_base):
    info = plsc.get_sparse_core_info()
    nw = info.num_cores * info.num_subcores
    nc = info.num_cores
    t_per_w = n_slice // nw
    tc = min(32, t_per_w)
    n_chunks = t_per_w // tc
    dh = d // 2
    n_groups = dh // _LANES

    mesh = plsc.VectorSubcoreMesh(core_axis_name="c", subcore_axis_name="s")

    @functools.partial(
        pl.kernel,
        mesh=mesh,
        compiler_params=pltpu.CompilerParams(needs_layout_passes=False),
        out_type=jax.ShapeDtypeStruct((n_slice, dh), jnp.int32),
        scratch_types=[
            pltpu.VMEM((tc,), jnp.int32),
            pltpu.VMEM((tc,), jnp.int32),
            pltpu.VMEM((tc, d), jnp.float32),
            pltpu.VMEM((tc, d), jnp.float32),
            pltpu.VMEM((tc, dh), jnp.int32),
            pltpu.VMEM((tc, dh), jnp.int32),
            pltpu.SemaphoreType.DMA,
            pltpu.SemaphoreType.DMA,
            pltpu.SemaphoreType.DMA,
            pltpu.SemaphoreType.DMA,
        ],
    )
    def gather_kernel(ids_hbm, word_hbm, out_hbm,
                      idx0, idx1, f0, f1, p0, p1,
                      gsem0, gsem1, wsem0, wsem1):
        wid = lax.axis_index("s") * nc + lax.axis_index("c")
        base = wid * t_per_w
        idx = (idx0, idx1)
        fbuf = (f0, f1)
        pbuf = (p0, p1)
        gsem = (gsem0, gsem1)
        wsem = (wsem0, wsem1)

        def pack_chunk(fb, pb):
            @plsc.parallel_loop(0, tc, 1, unroll=4)
            def _row(r):
                for g in range(n_groups):
                    a = plsc.bitcast(fb[r, pl.ds(g * _LANES, _LANES)],
                                     jnp.int32)
                    bb = plsc.bitcast(fb[r, pl.ds(dh + g * _LANES, _LANES)],
                                      jnp.int32)
                    pb[r, pl.ds(g * _LANES, _LANES)] = (
                        (((a + 0x8000) >> 16) & 0xFFFF)
                        | ((bb + 0x8000) & jnp.int32(-65536)))

        pltpu.sync_copy(ids_hbm.at[pl.ds(flat_base + base, tc)], idx[0])
        gathers = [pltpu.async_copy(word_hbm.at[idx[0]], fbuf[0], gsem[0])]
        writes = []
        for c in range(n_chunks):
            s = c % 2
            if c + 1 < n_chunks:
                sn = (c + 1) % 2
                pltpu.sync_copy(
                    ids_hbm.at[pl.ds(flat_base + base + (c + 1) * tc, tc)],
                    idx[sn])
                gathers.append(
                    pltpu.async_copy(word_hbm.at[idx[sn]], fbuf[sn], gsem[sn]))
            gathers[c].wait()
            if c >= 2:
                writes[c - 2].wait()
            pack_chunk(fbuf[s], pbuf[s])
            writes.append(
                pltpu.async_copy(
                    pbuf[s], out_hbm.at[pl.ds(base + c * tc, tc)], wsem[s]))
        for w in writes[max(0, n_chunks - 2):]:
            w.wait()

    return gather_kernel


def _ln_body(g_ref, p_ref, tt_ref, te_ref, gamma_ref, beta_ref, *rest):
    o_ref = rest[-1]
    gw = g_ref[...]
    lo = lax.bitcast_convert_type(gw << 16, jnp.float32)
    hi = lax.bitcast_convert_type(gw & jnp.int32(-65536), jnp.float32)
    g = jnp.concatenate([lo, hi], axis=-1)
    p = p_ref[...]
    t = tt_ref[...]
    te = te_ref[...]
    h = g + p[None] + te[0:1, :] + t * (te[1:2, :] - te[0:1, :])
    mu = jnp.mean(h, axis=-1, keepdims=True)
    var = jnp.mean((h - mu) ** 2, axis=-1, keepdims=True)
    o_ref[...] = ((h - mu) * lax.rsqrt(var + 1e-5)) * gamma_ref[...] + beta_ref[...]


def _sum_layernorm(gathered, pos_emb, tt_f, type_emb, gamma, beta, prev,
                   si, b, bs, blk):
    _, s, dh = gathered.shape
    d = dh * 2
    grid = (s // blk,)
    in_specs = [
        pl.BlockSpec((bs, blk, dh), lambda i: (0, i, 0)),
        pl.BlockSpec((blk, d), lambda i: (i, 0)),
        pl.BlockSpec((bs, blk, 1), lambda i, _si=si: (_si, i, 0)),
        pl.BlockSpec((2, d), lambda i: (0, 0)),
        pl.BlockSpec((1, d), lambda i: (0, 0)),
        pl.BlockSpec((1, d), lambda i: (0, 0)),
    ]
    args = [gathered, pos_emb, tt_f, type_emb, gamma, beta]
    aliases = {}
    if prev is not None:
        in_specs.append(pl.BlockSpec(memory_space=pl.ANY))
        args.append(prev)
        aliases = {6: 0}
    return pl.pallas_call(
        _ln_body,
        grid=grid,
        in_specs=in_specs,
        out_specs=pl.BlockSpec((bs, blk, d), lambda i, _si=si: (_si, i, 0)),
        out_shape=jax.ShapeDtypeStruct((b, s, d), jnp.float32),
        input_output_aliases=aliases,
    )(*args)


def kernel(input_ids, token_type_ids, word_emb, pos_emb, type_emb, gamma, beta):
    b, s = input_ids.shape
    vocab, d = word_emb.shape
    n = b * s
    bs = b // _NSLICES
    n_slice = bs * s

    ids_flat = input_ids.reshape(n)
    tt_f = token_type_ids.reshape(b, s, 1).astype(jnp.float32)
    gamma2 = gamma.reshape(1, d)
    beta2 = beta.reshape(1, d)

    gathered = [
        _make_sc_gather(d, n_slice, si * n_slice)(ids_flat, word_emb)
        for si in range(_NSLICES)
    ]

    out = None
    for si in range(_NSLICES):
        out = _sum_layernorm(
            gathered[si].reshape(bs, s, d // 2), pos_emb, tt_f, type_emb,
            gamma2, beta2, out, si, b, bs, blk=512,
        )
    return out


# speedup vs baseline: 1.6298x; 1.1312x over previous
import functools

import jax
import jax.numpy as jnp
from jax import lax
from jax.experimental import pallas as pl
from jax.experimental.pallas import tpu as pltpu
from jax.experimental.pallas import tpu_sc as plsc


def _make_sc_gather(d, b, s):
    info = plsc.get_sparse_core_info()
    nw = info.num_cores * info.num_subcores
    nc = info.num_cores
    n = b * s
    t_per_w = n // nw
    tc = 64
    n_chunks = t_per_w // tc
    w_per_row = s // t_per_w

    mesh = plsc.VectorSubcoreMesh(core_axis_name="c", subcore_axis_name="s")

    @functools.partial(
        pl.kernel,
        mesh=mesh,
        out_type=jax.ShapeDtypeStruct((n, d), jnp.float32),
        scratch_types=[
            pltpu.VMEM((tc,), jnp.int32),
            pltpu.VMEM((tc,), jnp.int32),
            pltpu.VMEM((tc, d), jnp.float32),
            pltpu.VMEM((tc, d), jnp.float32),
            pltpu.SemaphoreType.DMA,
            pltpu.SemaphoreType.DMA,
        ],
    )
    def gather_kernel(ids_hbm, word_hbm, out_hbm,
                      idx0, idx1, rows0, rows1, sem0, sem1):
        wid = lax.axis_index("s") * nc + lax.axis_index("c")
        row = wid // w_per_row
        s_off = (wid % w_per_row) * t_per_w
        base = wid * t_per_w
        idx = (idx0, idx1)
        rows = (rows0, rows1)
        sem = (sem0, sem1)
        pltpu.sync_copy(ids_hbm.at[row, pl.ds(s_off, tc)], idx[0])
        copies = [pltpu.async_copy(word_hbm.at[idx[0]], rows[0], sem[0])]
        for c in range(n_chunks):
            p = c % 2
            if c + 1 < n_chunks:
                pn = (c + 1) % 2
                pltpu.sync_copy(
                    ids_hbm.at[row, pl.ds(s_off + (c + 1) * tc, tc)], idx[pn])
                copies.append(
                    pltpu.async_copy(word_hbm.at[idx[pn]], rows[pn], sem[pn]))
            copies[c].wait()
            pltpu.sync_copy(rows[p], out_hbm.at[pl.ds(base + c * tc, tc)])

    return gather_kernel


def _ln_body(g_ref, p_ref, tt_ref, te_ref, gamma_ref, beta_ref, o_ref):
    g = g_ref[...]
    p = p_ref[...]
    t = tt_ref[...]
    te = te_ref[...]
    h = g + p[None] + te[0:1, :] + t * (te[1:2, :] - te[0:1, :])
    mu = jnp.mean(h, axis=-1, keepdims=True)
    var = jnp.mean((h - mu) ** 2, axis=-1, keepdims=True)
    o_ref[...] = ((h - mu) * lax.rsqrt(var + 1e-5)) * gamma_ref[...] + beta_ref[...]


def _sum_layernorm(gathered, pos_emb, tt_f, type_emb, gamma, beta, blk):
    b, s, d = gathered.shape
    grid = (s // blk,)
    return pl.pallas_call(
        _ln_body,
        grid=grid,
        in_specs=[
            pl.BlockSpec((b, blk, d), lambda i: (0, i, 0)),
            pl.BlockSpec((blk, d), lambda i: (i, 0)),
            pl.BlockSpec((b, blk, 1), lambda i: (0, i, 0)),
            pl.BlockSpec((2, d), lambda i: (0, 0)),
            pl.BlockSpec((1, d), lambda i: (0, 0)),
            pl.BlockSpec((1, d), lambda i: (0, 0)),
        ],
        out_specs=pl.BlockSpec((b, blk, d), lambda i: (0, i, 0)),
        out_shape=jax.ShapeDtypeStruct((b, s, d), jnp.float32),
    )(gathered, pos_emb, tt_f, type_emb, gamma, beta)


def kernel(input_ids, token_type_ids, word_emb, pos_emb, type_emb, gamma, beta):
    b, s = input_ids.shape
    vocab, d = word_emb.shape

    gathered = _make_sc_gather(d, b, s)(input_ids, word_emb)

    tt_f = token_type_ids.reshape(b, s, 1).astype(jnp.float32)
    out = _sum_layernorm(
        gathered.reshape(b, s, d), pos_emb, tt_f, type_emb,
        gamma.reshape(1, d), beta.reshape(1, d), blk=256,
    )
    return out
